# baseline (speedup 1.0000x reference)
import functools

import jax
import jax.numpy as jnp
from jax import lax
from jax.experimental import pallas as pl
from jax.experimental.pallas import tpu as pltpu
from jax.experimental.pallas import tpu_sc as plsc

HIDDEN = 3072
SEQ = 4096
NUM_TOKENS = 2 * SEQ
NC = 2
NS = 16
NW = NC * NS
PER_W = NUM_TOKENS // NW
CHUNK = 16
NCHUNK = PER_W // CHUNK
NBUF = 2

_mesh = plsc.VectorSubcoreMesh(core_axis_name="c", subcore_axis_name="s")


@functools.partial(
    pl.kernel,
    mesh=_mesh,
    out_type=jax.ShapeDtypeStruct((2, SEQ, HIDDEN), jnp.float32),
    scratch_types=[
        pltpu.VMEM((PER_W,), jnp.int32),
        pltpu.VMEM((NBUF * CHUNK,), jnp.int32),
        pltpu.VMEM((NBUF, CHUNK, HIDDEN), jnp.float32),
        pltpu.SemaphoreType.DMA,
        pltpu.SemaphoreType.DMA,
        pltpu.SemaphoreType.DMA,
        pltpu.SemaphoreType.DMA,
        pltpu.SemaphoreType.DMA,
    ],
)
def _embed_lookup(
    table_hbm, ids_hbm, out_hbm, idx_v, hidx_v, rows_v, si0, si1, so0, so1, ssem
):
    in_sem = (si0, si1)
    out_sem = (so0, so1)
    wid = lax.axis_index("s") * NC + lax.axis_index("c")
    tiles_per_row = SEQ // PER_W
    brow = wid // tiles_per_row
    bcol = (wid % tiles_per_row) * PER_W
    head = NBUF * CHUNK
    full_cp = pltpu.async_copy(ids_hbm.at[brow, pl.ds(bcol, PER_W)], idx_v, ssem)
    pltpu.sync_copy(ids_hbm.at[brow, pl.ds(bcol, head)], hidx_v)

    def gather(j, b):
        return pltpu.async_copy(
            table_hbm.at[idx_v.at[pl.ds(j * CHUNK, CHUNK)]], rows_v.at[b], in_sem[b]
        )

    def put(j, b):
        return pltpu.async_copy(
            rows_v.at[b], out_hbm.at[brow, pl.ds(bcol + j * CHUNK, CHUNK)], out_sem[b]
        )

    def gather_head(j, b):
        return pltpu.async_copy(
            table_hbm.at[hidx_v.at[pl.ds(j * CHUNK, CHUNK)]], rows_v.at[b], in_sem[b]
        )

    gcp = [gather_head(0, 0), gather_head(1, 1)]
    full_cp.wait()
    pcp = [None, None]
    for j in range(NCHUNK):
        b = j % NBUF
        gcp[b].wait()
        pcp[b] = put(j, b)
        if j + NBUF < NCHUNK:
            pcp[b].wait()
            gcp[b] = gather(j + NBUF, b)
    pcp[0].wait()
    pcp[1].wait()


def kernel(embed_weight, input_ids):
    return _embed_lookup(embed_weight, input_ids.astype(jnp.int32))


# speedup vs baseline: 1.0108x; 1.0108x over previous
import functools

import jax
import jax.numpy as jnp
from jax import lax
from jax.experimental import pallas as pl
from jax.experimental.pallas import tpu as pltpu
from jax.experimental.pallas import tpu_sc as plsc

HIDDEN = 3072
SEQ = 4096
NUM_TOKENS = 2 * SEQ
NC = 2
NS = 16
NW = NC * NS
PER_W = NUM_TOKENS // NW
CHUNK = 16
SIZES = [8] + [16] * 15 + [8]
OFFS = [sum(SIZES[:k]) for k in range(len(SIZES))]
NBUF = 2

_mesh = plsc.VectorSubcoreMesh(core_axis_name="c", subcore_axis_name="s")


@functools.partial(
    pl.kernel,
    mesh=_mesh,
    out_type=jax.ShapeDtypeStruct((2, SEQ, HIDDEN), jnp.float32),
    scratch_types=[
        pltpu.VMEM((PER_W,), jnp.int32),
        pltpu.VMEM((NBUF, CHUNK, HIDDEN), jnp.float32),
        pltpu.SemaphoreType.DMA,
        pltpu.SemaphoreType.DMA,
        pltpu.SemaphoreType.DMA,
        pltpu.SemaphoreType.DMA,
    ],
)
def _embed_lookup(table_hbm, ids_hbm, out_hbm, idx_v, rows_v, si0, si1, so0, so1):
    in_sem = (si0, si1)
    out_sem = (so0, so1)
    wid = lax.axis_index("s") * NC + lax.axis_index("c")
    tiles_per_row = SEQ // PER_W
    brow = wid // tiles_per_row
    bcol = (wid % tiles_per_row) * PER_W
    pltpu.sync_copy(ids_hbm.at[brow, pl.ds(bcol, PER_W)], idx_v)

    def gather(j, b):
        return pltpu.async_copy(
            table_hbm.at[idx_v.at[pl.ds(OFFS[j], SIZES[j])]],
            rows_v.at[b, pl.ds(0, SIZES[j])],
            in_sem[b],
        )

    def put(j, b):
        return pltpu.async_copy(
            rows_v.at[b, pl.ds(0, SIZES[j])],
            out_hbm.at[brow, pl.ds(bcol + OFFS[j], SIZES[j])],
            out_sem[b],
        )

    nchunk = len(SIZES)
    gcp = [gather(0, 0), gather(1, 1)]
    pcp = [None, None]
    for j in range(nchunk):
        b = j % NBUF
        gcp[b].wait()
        pcp[b] = put(j, b)
        if j + NBUF < nchunk:
            pcp[b].wait()
            gcp[b] = gather(j + NBUF, b)
    pcp[0].wait()
    pcp[1].wait()


def kernel(embed_weight, input_ids):
    return _embed_lookup(embed_weight, input_ids.astype(jnp.int32))
